# baseline (speedup 1.0000x reference)
import functools

import jax
import jax.numpy as jnp
from jax import lax
from jax.experimental import pallas as pl
from jax.experimental.pallas import tpu as pltpu
from jax.experimental.pallas import tpu_sc as plsc

_B = 16384
_D = 64
_NW = 32
_BPW = _B // _NW
_NCH = 4
_CH = _BPW // _NCH


def _body(eidx_hbm, pidx_hbm, uidx_hbm, node_hbm, edge_hbm, out_hbm,
          eidx_v, pidx_v, uidx_v, e_rows, p_rows, u_rows, out_v, sem):
    c = lax.axis_index("c")
    s = lax.axis_index("s")
    wid = s * 2 + c

    pltpu.sync_copy(eidx_hbm.at[wid], eidx_v)
    pltpu.sync_copy(pidx_hbm.at[wid], pidx_v)
    pltpu.sync_copy(uidx_hbm.at[wid], uidx_v)

    copies = []
    for j in range(_NCH):
        dst = pl.ds(j * _CH, _CH)
        copies.append(pltpu.async_copy(node_hbm.at[eidx_v.at[j]],
                                       e_rows.at[dst], sem))
        copies.append(pltpu.async_copy(edge_hbm.at[pidx_v.at[j]],
                                       p_rows.at[dst], sem))
        copies.append(pltpu.async_copy(node_hbm.at[uidx_v.at[j]],
                                       u_rows.at[dst], sem))
    for cp in copies:
        cp.wait()

    iota = lax.iota(jnp.int32, 16)

    def g_body(g, carry):
        rows = g * 16 + iota

        def j_body(j, accs):
            a0, a1, a2, a3 = accs
            new = []
            for k in range(4):
                colv = jnp.full((16,), j * 4 + k, jnp.int32)
                ev = plsc.load_gather(e_rows, [rows, colv])
                pv = plsc.load_gather(p_rows, [rows, colv])
                uv = plsc.load_gather(u_rows, [rows, colv])
                new.append(ev * pv * uv)
            return (a0 + new[0], a1 + new[1], a2 + new[2], a3 + new[3])

        z = jnp.zeros((16,), jnp.float32)
        a0, a1, a2, a3 = lax.fori_loop(0, _D // 4, j_body, (z, z, z, z))
        out_v[pl.ds(g * 16, 16)] = (a0 + a1) + (a2 + a3)
        return carry

    lax.fori_loop(0, _BPW // 16, g_body, 0)

    pltpu.sync_copy(out_v, out_hbm.at[pl.ds(wid * _BPW, _BPW)])


def _distmult(eidx, pidx, uidx, node, edge):
    mesh = plsc.VectorSubcoreMesh(core_axis_name="c", subcore_axis_name="s")
    fn = functools.partial(
        pl.kernel,
        out_type=jax.ShapeDtypeStruct((_B,), jnp.float32),
        mesh=mesh,
        compiler_params=pltpu.CompilerParams(
            needs_layout_passes=False, use_tc_tiling_on_sc=False),
        scratch_types=[
            pltpu.VMEM((_NCH, _CH), jnp.int32),
            pltpu.VMEM((_NCH, _CH), jnp.int32),
            pltpu.VMEM((_NCH, _CH), jnp.int32),
            pltpu.VMEM((_BPW, _D), jnp.float32),
            pltpu.VMEM((_BPW, _D), jnp.float32),
            pltpu.VMEM((_BPW, _D), jnp.float32),
            pltpu.VMEM((_BPW,), jnp.float32),
            pltpu.SemaphoreType.DMA,
        ],
    )(_body)
    return fn(eidx, pidx, uidx, node, edge)


def kernel(e_idc, p_idc, u_idc, node_embeddings, edge_embeddings):
    e = e_idc.astype(jnp.int32).reshape(_NW, _NCH, _CH)
    p = p_idc.astype(jnp.int32).reshape(_NW, _NCH, _CH)
    u = u_idc.astype(jnp.int32).reshape(_NW, _NCH, _CH)
    return _distmult(e, p, u, node_embeddings, edge_embeddings)


# speedup vs baseline: 1.1228x; 1.1228x over previous
import functools

import jax
import jax.numpy as jnp
from jax import lax
from jax.experimental import pallas as pl
from jax.experimental.pallas import tpu as pltpu
from jax.experimental.pallas import tpu_sc as plsc

_B = 16384
_N = 1000000
_PK = 524288
_R = 1000
_D = 64
_NW = 32
_BPW = _B // _NW
_CH = 64
_NCH = _BPW // _CH
_TBLK = 4096
_TGRID = _PK // _TBLK


def _tpack_body(x1_ref, x2_ref, o_ref):
    i64 = jnp.eye(_D, dtype=jnp.float32)
    dn = (((0,), (0,)), ((), ()))
    y1 = lax.dot_general(x1_ref[...], i64, dn,
                         precision=lax.Precision.HIGHEST,
                         preferred_element_type=jnp.float32)
    y2 = lax.dot_general(x2_ref[...], i64, dn,
                         precision=lax.Precision.HIGHEST,
                         preferred_element_type=jnp.float32)
    o_ref[...] = jnp.concatenate([y1, y2], axis=1)


def _tpack(node_t):
    return pl.pallas_call(
        _tpack_body,
        grid=(_TGRID,),
        in_specs=[
            pl.BlockSpec((_D, _TBLK), lambda i: (0, i)),
            pl.BlockSpec(
                (_D, _TBLK),
                lambda i: (0, jnp.minimum(i + _TGRID, _N // _TBLK))),
        ],
        out_specs=pl.BlockSpec((_TBLK, 128), lambda i: (i, 0)),
        out_shape=jax.ShapeDtypeStruct((_PK, 128), jnp.float32),
    )(node_t, node_t)


def _sc_body(eg_hbm, eh_hbm, ug_hbm, uh_hbm, pi_hbm, node2, edge_t, out_hbm,
             egv, ehv, ugv, uhv, piv, edge_v,
             ebufa, ubufa, ebufb, ubufb,
             out_v, sema, semb):
    c = lax.axis_index("c")
    s = lax.axis_index("s")
    wid = s * 2 + c
    base = wid * _BPW

    pltpu.sync_copy(eg_hbm.at[wid], egv)
    pltpu.sync_copy(eh_hbm.at[wid], ehv)
    pltpu.sync_copy(ug_hbm.at[wid], ugv)
    pltpu.sync_copy(uh_hbm.at[wid], uhv)
    pltpu.sync_copy(pi_hbm.at[wid], piv)
    pltpu.sync_copy(edge_t, edge_v)

    bufs = [(ebufa, ubufa, sema), (ebufb, ubufb, semb)]

    def issue(j):
        eb_, ub_, sem = bufs[j % 2]
        return [
            pltpu.async_copy(node2.at[egv.at[j, 0]], eb_, sem),
            pltpu.async_copy(node2.at[ugv.at[j, 0]], ub_, sem),
        ]

    iota = lax.iota(jnp.int32, 16)
    jvecs = [jnp.full((16,), j, jnp.int32) for j in range(_NCH)]
    zero16 = jnp.zeros((16,), jnp.int32)

    def compute(j):
        eb_, ub_, _ = bufs[j % 2]
        jvec = jvecs[j]
        for g in range(_CH // 16):
            rows = g * 16 + iota
            he = plsc.load_gather(ehv, [jvec, zero16, rows])
            hu = plsc.load_gather(uhv, [jvec, zero16, rows])
            pp = plsc.load_gather(piv, [jvec, zero16, rows])
            z = jnp.zeros((16,), jnp.float32)

            def jj_body(jj, accs):
                a0, a1, a2, a3 = accs
                new = []
                for k in range(4):
                    f = jj * 4 + k
                    fv = jnp.full((16,), f, jnp.int32)
                    ev = plsc.load_gather(eb_, [rows, he + f])
                    uv = plsc.load_gather(ub_, [rows, hu + f])
                    pv = plsc.load_gather(edge_v, [fv, pp])
                    new.append(ev * pv * uv)
                return (a0 + new[0], a1 + new[1], a2 + new[2], a3 + new[3])

            a0, a1, a2, a3 = lax.fori_loop(0, _D // 4, jj_body, (z, z, z, z))
            out_v[pl.ds(j * _CH + g * 16, 16)] = (a0 + a1) + (a2 + a3)

    prev = issue(0)
    for j in range(1, _NCH):
        nxt = issue(j)
        for cp in prev:
            cp.wait()
        compute(j - 1)
        prev = nxt
    for cp in prev:
        cp.wait()
    compute(_NCH - 1)

    pltpu.sync_copy(out_v, out_hbm.at[pl.ds(base, _BPW)])


def _sc_call(eg, eh, ug, uh, pi, node2, edge_t):
    mesh = plsc.VectorSubcoreMesh(core_axis_name="c", subcore_axis_name="s")
    fn = functools.partial(
        pl.kernel,
        out_type=jax.ShapeDtypeStruct((_B,), jnp.float32),
        mesh=mesh,
        compiler_params=pltpu.CompilerParams(needs_layout_passes=False),
        scratch_types=[
            pltpu.VMEM((_NCH, 1, _CH), jnp.int32),
            pltpu.VMEM((_NCH, 1, _CH), jnp.int32),
            pltpu.VMEM((_NCH, 1, _CH), jnp.int32),
            pltpu.VMEM((_NCH, 1, _CH), jnp.int32),
            pltpu.VMEM((_NCH, 1, _CH), jnp.int32),
            pltpu.VMEM((_D, _R), jnp.float32),
            pltpu.VMEM((_CH, 128), jnp.float32),
            pltpu.VMEM((_CH, 128), jnp.float32),
            pltpu.VMEM((_CH, 128), jnp.float32),
            pltpu.VMEM((_CH, 128), jnp.float32),
            pltpu.VMEM((_BPW,), jnp.float32),
            pltpu.SemaphoreType.DMA,
            pltpu.SemaphoreType.DMA,
        ],
    )(_sc_body)
    return fn(eg, eh, ug, uh, pi, node2, edge_t)


def kernel(e_idc, p_idc, u_idc, node_embeddings, edge_embeddings):
    e = e_idc.astype(jnp.int32)
    p = p_idc.astype(jnp.int32)
    u = u_idc.astype(jnp.int32)
    shp = (_NW, _NCH, 1, _CH)
    eg = jnp.where(e < _PK, e, e - _PK).reshape(shp)
    eh = jnp.where(e < _PK, 0, _D).astype(jnp.int32).reshape(shp)
    ug = jnp.where(u < _PK, u, u - _PK).reshape(shp)
    uh = jnp.where(u < _PK, 0, _D).astype(jnp.int32).reshape(shp)
    pi = p.reshape(shp)
    node2 = _tpack(node_embeddings.T)
    return _sc_call(eg, eh, ug, uh, pi, node2, edge_embeddings.T)


# speedup vs baseline: 1.9798x; 1.7632x over previous
import functools

import jax
import jax.numpy as jnp
from jax import lax
from jax.experimental import pallas as pl
from jax.experimental.pallas import tpu as pltpu
from jax.experimental.pallas import tpu_sc as plsc

_B = 16384
_N = 1000000
_PK = 524288
_R = 1000
_D = 64
_NW = 32
_BPW = _B // _NW
_CH = 64
_NCH = _BPW // _CH
_TBLK = 4096
_TGRID = _PK // _TBLK


def _tpack_body(x1_ref, x2_ref, o_ref):
    o_ref[:, 0:_D] = x1_ref[...].T
    o_ref[:, _D:128] = x2_ref[...].T


def _tpack(node_t):
    return pl.pallas_call(
        _tpack_body,
        grid=(_TGRID,),
        in_specs=[
            pl.BlockSpec((_D, _TBLK), lambda i: (0, i)),
            pl.BlockSpec(
                (_D, _TBLK),
                lambda i: (0, jnp.minimum(i + _TGRID, _N // _TBLK))),
        ],
        out_specs=pl.BlockSpec((_TBLK, 128), lambda i: (i, 0)),
        out_shape=jax.ShapeDtypeStruct((_PK, 128), jnp.float32),
    )(node_t, node_t)


def _sc_body(eg_hbm, eh_hbm, ug_hbm, uh_hbm, pi_hbm, node2, edge_t, out_hbm,
             egv, ehv, ugv, uhv, piv, edge_v,
             ebufa, ubufa, ebufb, ubufb,
             out_v, sema, semb):
    c = lax.axis_index("c")
    s = lax.axis_index("s")
    wid = s * 2 + c
    base = wid * _BPW

    pltpu.sync_copy(eg_hbm.at[wid], egv)
    pltpu.sync_copy(eh_hbm.at[wid], ehv)
    pltpu.sync_copy(ug_hbm.at[wid], ugv)
    pltpu.sync_copy(uh_hbm.at[wid], uhv)
    pltpu.sync_copy(pi_hbm.at[wid], piv)
    pltpu.sync_copy(edge_t, edge_v)

    bufs = [(ebufa, ubufa, sema), (ebufb, ubufb, semb)]

    def issue(j):
        eb_, ub_, sem = bufs[j % 2]
        return [
            pltpu.async_copy(node2.at[egv.at[j, 0]], eb_, sem),
            pltpu.async_copy(node2.at[ugv.at[j, 0]], ub_, sem),
        ]

    iota = lax.iota(jnp.int32, 16)
    jvecs = [jnp.full((16,), j, jnp.int32) for j in range(_NCH)]
    zero16 = jnp.zeros((16,), jnp.int32)

    def compute(j):
        eb_, ub_, _ = bufs[j % 2]
        jvec = jvecs[j]
        for g in range(_CH // 16):
            rows = g * 16 + iota
            he = plsc.load_gather(ehv, [jvec, zero16, rows])
            hu = plsc.load_gather(uhv, [jvec, zero16, rows])
            pp = plsc.load_gather(piv, [jvec, zero16, rows])
            z = jnp.zeros((16,), jnp.float32)

            def jj_body(jj, accs):
                a0, a1, a2, a3 = accs
                new = []
                for k in range(4):
                    f = jj * 4 + k
                    fv = jnp.full((16,), f, jnp.int32)
                    ev = plsc.load_gather(eb_, [rows, he + f])
                    uv = plsc.load_gather(ub_, [rows, hu + f])
                    pv = plsc.load_gather(edge_v, [fv, pp])
                    new.append(ev * pv * uv)
                return (a0 + new[0], a1 + new[1], a2 + new[2], a3 + new[3])

            a0, a1, a2, a3 = lax.fori_loop(0, _D // 4, jj_body, (z, z, z, z))
            out_v[pl.ds(j * _CH + g * 16, 16)] = (a0 + a1) + (a2 + a3)

    prev = issue(0)
    for j in range(1, _NCH):
        nxt = issue(j)
        for cp in prev:
            cp.wait()
        compute(j - 1)
        prev = nxt
    for cp in prev:
        cp.wait()
    compute(_NCH - 1)

    pltpu.sync_copy(out_v, out_hbm.at[pl.ds(base, _BPW)])


def _sc_call(eg, eh, ug, uh, pi, node2, edge_t):
    mesh = plsc.VectorSubcoreMesh(core_axis_name="c", subcore_axis_name="s")
    fn = functools.partial(
        pl.kernel,
        out_type=jax.ShapeDtypeStruct((_B,), jnp.float32),
        mesh=mesh,
        compiler_params=pltpu.CompilerParams(needs_layout_passes=False),
        scratch_types=[
            pltpu.VMEM((_NCH, 1, _CH), jnp.int32),
            pltpu.VMEM((_NCH, 1, _CH), jnp.int32),
            pltpu.VMEM((_NCH, 1, _CH), jnp.int32),
            pltpu.VMEM((_NCH, 1, _CH), jnp.int32),
            pltpu.VMEM((_NCH, 1, _CH), jnp.int32),
            pltpu.VMEM((_D, _R), jnp.float32),
            pltpu.VMEM((_CH, 128), jnp.float32),
            pltpu.VMEM((_CH, 128), jnp.float32),
            pltpu.VMEM((_CH, 128), jnp.float32),
            pltpu.VMEM((_CH, 128), jnp.float32),
            pltpu.VMEM((_BPW,), jnp.float32),
            pltpu.SemaphoreType.DMA,
            pltpu.SemaphoreType.DMA,
        ],
    )(_sc_body)
    return fn(eg, eh, ug, uh, pi, node2, edge_t)


def kernel(e_idc, p_idc, u_idc, node_embeddings, edge_embeddings):
    e = e_idc.astype(jnp.int32)
    p = p_idc.astype(jnp.int32)
    u = u_idc.astype(jnp.int32)
    shp = (_NW, _NCH, 1, _CH)
    eg = jnp.where(e < _PK, e, e - _PK).reshape(shp)
    eh = jnp.where(e < _PK, 0, _D).astype(jnp.int32).reshape(shp)
    ug = jnp.where(u < _PK, u, u - _PK).reshape(shp)
    uh = jnp.where(u < _PK, 0, _D).astype(jnp.int32).reshape(shp)
    pi = p.reshape(shp)
    node2 = _tpack(node_embeddings.T)
    return _sc_call(eg, eh, ug, uh, pi, node2, edge_embeddings.T)


# speedup vs baseline: 2.4521x; 1.2386x over previous
import functools

import jax
import jax.numpy as jnp
from jax import lax
from jax.experimental import pallas as pl
from jax.experimental.pallas import tpu as pltpu
from jax.experimental.pallas import tpu_sc as plsc

_B = 16384
_N = 1000000
_PK = 524288
_R = 1000
_D = 64
_NW = 32
_BPW = _B // _NW
_CH = 64
_NCH = _BPW // _CH
_TBLK = 4096
_TGRID = _PK // _TBLK


def _tpack_body(x1_ref, x2_ref, o_ref):
    xb = jnp.concatenate([x1_ref[...], x2_ref[...]], axis=0)
    o_ref[...] = xb.T


def _tpack(node_t):
    return pl.pallas_call(
        _tpack_body,
        grid=(_TGRID,),
        in_specs=[
            pl.BlockSpec((_D, _TBLK), lambda i: (0, i)),
            pl.BlockSpec(
                (_D, _TBLK),
                lambda i: (0, jnp.minimum(i + _TGRID, _N // _TBLK))),
        ],
        out_specs=pl.BlockSpec((_TBLK, 128), lambda i: (i, 0)),
        out_shape=jax.ShapeDtypeStruct((_PK, 128), jnp.float32),
    )(node_t, node_t)


def _sc_body(eg_hbm, eh_hbm, ug_hbm, uh_hbm, pi_hbm, node2, edge_t, out_hbm,
             egv, ehv, ugv, uhv, piv, edge_v,
             ebufa, ubufa, ebufb, ubufb,
             out_v, sema, semb):
    c = lax.axis_index("c")
    s = lax.axis_index("s")
    wid = s * 2 + c
    base = wid * _BPW

    pltpu.sync_copy(eg_hbm.at[wid], egv)
    pltpu.sync_copy(eh_hbm.at[wid], ehv)
    pltpu.sync_copy(ug_hbm.at[wid], ugv)
    pltpu.sync_copy(uh_hbm.at[wid], uhv)
    pltpu.sync_copy(pi_hbm.at[wid], piv)
    pltpu.sync_copy(edge_t, edge_v)

    bufs = [(ebufa, ubufa, sema), (ebufb, ubufb, semb)]

    def issue(j):
        eb_, ub_, sem = bufs[j % 2]
        return [
            pltpu.async_copy(node2.at[egv.at[j, 0]], eb_, sem),
            pltpu.async_copy(node2.at[ugv.at[j, 0]], ub_, sem),
        ]

    iota = lax.iota(jnp.int32, 16)
    jvecs = [jnp.full((16,), j, jnp.int32) for j in range(_NCH)]
    zero16 = jnp.zeros((16,), jnp.int32)

    def compute(j):
        eb_, ub_, _ = bufs[j % 2]
        jvec = jvecs[j]
        for g in range(_CH // 16):
            rows = g * 16 + iota
            he = plsc.load_gather(ehv, [jvec, zero16, rows])
            hu = plsc.load_gather(uhv, [jvec, zero16, rows])
            pp = plsc.load_gather(piv, [jvec, zero16, rows])
            z = jnp.zeros((16,), jnp.float32)

            def jj_body(jj, accs):
                a0, a1, a2, a3 = accs
                new = []
                for k in range(4):
                    f = jj * 4 + k
                    fv = jnp.full((16,), f, jnp.int32)
                    ev = plsc.load_gather(eb_, [rows, he + f])
                    uv = plsc.load_gather(ub_, [rows, hu + f])
                    pv = plsc.load_gather(edge_v, [fv, pp])
                    new.append(ev * pv * uv)
                return (a0 + new[0], a1 + new[1], a2 + new[2], a3 + new[3])

            a0, a1, a2, a3 = lax.fori_loop(0, _D // 4, jj_body, (z, z, z, z))
            out_v[pl.ds(j * _CH + g * 16, 16)] = (a0 + a1) + (a2 + a3)

    prev = issue(0)
    for j in range(1, _NCH):
        nxt = issue(j)
        for cp in prev:
            cp.wait()
        compute(j - 1)
        prev = nxt
    for cp in prev:
        cp.wait()
    compute(_NCH - 1)

    pltpu.sync_copy(out_v, out_hbm.at[pl.ds(base, _BPW)])


def _sc_call(eg, eh, ug, uh, pi, node2, edge_t):
    mesh = plsc.VectorSubcoreMesh(core_axis_name="c", subcore_axis_name="s")
    fn = functools.partial(
        pl.kernel,
        out_type=jax.ShapeDtypeStruct((_B,), jnp.float32),
        mesh=mesh,
        compiler_params=pltpu.CompilerParams(needs_layout_passes=False),
        scratch_types=[
            pltpu.VMEM((_NCH, 1, _CH), jnp.int32),
            pltpu.VMEM((_NCH, 1, _CH), jnp.int32),
            pltpu.VMEM((_NCH, 1, _CH), jnp.int32),
            pltpu.VMEM((_NCH, 1, _CH), jnp.int32),
            pltpu.VMEM((_NCH, 1, _CH), jnp.int32),
            pltpu.VMEM((_D, _R), jnp.float32),
            pltpu.VMEM((_CH, 128), jnp.float32),
            pltpu.VMEM((_CH, 128), jnp.float32),
            pltpu.VMEM((_CH, 128), jnp.float32),
            pltpu.VMEM((_CH, 128), jnp.float32),
            pltpu.VMEM((_BPW,), jnp.float32),
            pltpu.SemaphoreType.DMA,
            pltpu.SemaphoreType.DMA,
        ],
    )(_sc_body)
    return fn(eg, eh, ug, uh, pi, node2, edge_t)


def kernel(e_idc, p_idc, u_idc, node_embeddings, edge_embeddings):
    e = e_idc.astype(jnp.int32)
    p = p_idc.astype(jnp.int32)
    u = u_idc.astype(jnp.int32)
    shp = (_NW, _NCH, 1, _CH)
    eg = jnp.where(e < _PK, e, e - _PK).reshape(shp)
    eh = jnp.where(e < _PK, 0, _D).astype(jnp.int32).reshape(shp)
    ug = jnp.where(u < _PK, u, u - _PK).reshape(shp)
    uh = jnp.where(u < _PK, 0, _D).astype(jnp.int32).reshape(shp)
    pi = p.reshape(shp)
    node2 = _tpack(node_embeddings.T)
    return _sc_call(eg, eh, ug, uh, pi, node2, edge_embeddings.T)


# speedup vs baseline: 2.7540x; 1.1232x over previous
import functools

import jax
import jax.numpy as jnp
from jax import lax
from jax.experimental import pallas as pl
from jax.experimental.pallas import tpu as pltpu
from jax.experimental.pallas import tpu_sc as plsc

_B = 16384
_N = 1000000
_PK = 524288
_R = 1000
_D = 64
_NW = 32
_BPW = _B // _NW
_CH = 64
_NCH = _BPW // _CH
_TBLK = 8192
_TGRID = _PK // _TBLK


def _tpack_body(x1_ref, x2_ref, o_ref):
    xb = jnp.concatenate([x1_ref[...], x2_ref[...]], axis=0)
    o_ref[...] = xb.T


def _tpack(node_t):
    return pl.pallas_call(
        _tpack_body,
        grid=(_TGRID,),
        in_specs=[
            pl.BlockSpec((_D, _TBLK), lambda i: (0, i)),
            pl.BlockSpec(
                (_D, _TBLK),
                lambda i: (0, jnp.minimum(i + _TGRID, _N // _TBLK))),
        ],
        out_specs=pl.BlockSpec((_TBLK, 128), lambda i: (i, 0)),
        out_shape=jax.ShapeDtypeStruct((_PK, 128), jnp.float32),
    )(node_t, node_t)


def _sc_body(eg_hbm, eh_hbm, ug_hbm, uh_hbm, pi_hbm, node2, edge_t, out_hbm,
             egv, ehv, ugv, uhv, piv, edge_v,
             ebufa, ubufa, ebufb, ubufb,
             out_v, sema, semb):
    c = lax.axis_index("c")
    s = lax.axis_index("s")
    wid = s * 2 + c
    base = wid * _BPW

    pltpu.sync_copy(eg_hbm.at[wid], egv)
    pltpu.sync_copy(eh_hbm.at[wid], ehv)
    pltpu.sync_copy(ug_hbm.at[wid], ugv)
    pltpu.sync_copy(uh_hbm.at[wid], uhv)
    pltpu.sync_copy(pi_hbm.at[wid], piv)
    pltpu.sync_copy(edge_t, edge_v)

    bufs = [(ebufa, ubufa, sema), (ebufb, ubufb, semb)]

    def issue(j):
        eb_, ub_, sem = bufs[j % 2]
        return [
            pltpu.async_copy(node2.at[egv.at[j, 0]], eb_, sem),
            pltpu.async_copy(node2.at[ugv.at[j, 0]], ub_, sem),
        ]

    iota = lax.iota(jnp.int32, 16)
    jvecs = [jnp.full((16,), j, jnp.int32) for j in range(_NCH)]
    zero16 = jnp.zeros((16,), jnp.int32)

    def compute(j):
        eb_, ub_, _ = bufs[j % 2]
        jvec = jvecs[j]
        for g in range(_CH // 16):
            rows = g * 16 + iota
            he = plsc.load_gather(ehv, [jvec, zero16, rows])
            hu = plsc.load_gather(uhv, [jvec, zero16, rows])
            pp = plsc.load_gather(piv, [jvec, zero16, rows])
            z = jnp.zeros((16,), jnp.float32)

            def jj_body(jj, accs):
                a0, a1, a2, a3 = accs
                new = []
                for k in range(4):
                    f = jj * 4 + k
                    fv = jnp.full((16,), f, jnp.int32)
                    ev = plsc.load_gather(eb_, [rows, he + f])
                    uv = plsc.load_gather(ub_, [rows, hu + f])
                    pv = plsc.load_gather(edge_v, [fv, pp])
                    new.append(ev * pv * uv)
                return (a0 + new[0], a1 + new[1], a2 + new[2], a3 + new[3])

            a0, a1, a2, a3 = lax.fori_loop(0, _D // 4, jj_body, (z, z, z, z))
            out_v[pl.ds(j * _CH + g * 16, 16)] = (a0 + a1) + (a2 + a3)

    prev = issue(0)
    for j in range(1, _NCH):
        nxt = issue(j)
        for cp in prev:
            cp.wait()
        compute(j - 1)
        prev = nxt
    for cp in prev:
        cp.wait()
    compute(_NCH - 1)

    pltpu.sync_copy(out_v, out_hbm.at[pl.ds(base, _BPW)])


def _sc_call(eg, eh, ug, uh, pi, node2, edge_t):
    mesh = plsc.VectorSubcoreMesh(core_axis_name="c", subcore_axis_name="s")
    fn = functools.partial(
        pl.kernel,
        out_type=jax.ShapeDtypeStruct((_B,), jnp.float32),
        mesh=mesh,
        compiler_params=pltpu.CompilerParams(needs_layout_passes=False),
        scratch_types=[
            pltpu.VMEM((_NCH, 1, _CH), jnp.int32),
            pltpu.VMEM((_NCH, 1, _CH), jnp.int32),
            pltpu.VMEM((_NCH, 1, _CH), jnp.int32),
            pltpu.VMEM((_NCH, 1, _CH), jnp.int32),
            pltpu.VMEM((_NCH, 1, _CH), jnp.int32),
            pltpu.VMEM((_D, _R), jnp.float32),
            pltpu.VMEM((_CH, 128), jnp.float32),
            pltpu.VMEM((_CH, 128), jnp.float32),
            pltpu.VMEM((_CH, 128), jnp.float32),
            pltpu.VMEM((_CH, 128), jnp.float32),
            pltpu.VMEM((_BPW,), jnp.float32),
            pltpu.SemaphoreType.DMA,
            pltpu.SemaphoreType.DMA,
        ],
    )(_sc_body)
    return fn(eg, eh, ug, uh, pi, node2, edge_t)


def kernel(e_idc, p_idc, u_idc, node_embeddings, edge_embeddings):
    e = e_idc.astype(jnp.int32)
    p = p_idc.astype(jnp.int32)
    u = u_idc.astype(jnp.int32)
    shp = (_NW, _NCH, 1, _CH)
    eg = jnp.where(e < _PK, e, e - _PK).reshape(shp)
    eh = jnp.where(e < _PK, 0, _D).astype(jnp.int32).reshape(shp)
    ug = jnp.where(u < _PK, u, u - _PK).reshape(shp)
    uh = jnp.where(u < _PK, 0, _D).astype(jnp.int32).reshape(shp)
    pi = p.reshape(shp)
    node2 = _tpack(node_embeddings.T)
    return _sc_call(eg, eh, ug, uh, pi, node2, edge_embeddings.T)


# speedup vs baseline: 2.8219x; 1.0246x over previous
import functools

import jax
import jax.numpy as jnp
from jax import lax
from jax.experimental import pallas as pl
from jax.experimental.pallas import tpu as pltpu
from jax.experimental.pallas import tpu_sc as plsc

_B = 16384
_N = 1000000
_PK = 524288
_R = 1000
_D = 64
_NW = 32
_BPW = _B // _NW
_CH = 64
_NCH = _BPW // _CH
_TBLK = 16384
_TGRID = _PK // _TBLK


def _tpack_body(x1_ref, x2_ref, o_ref):
    xb = jnp.concatenate([x1_ref[...], x2_ref[...]], axis=0)
    o_ref[...] = xb.T


def _tpack(node_t):
    return pl.pallas_call(
        _tpack_body,
        grid=(_TGRID,),
        in_specs=[
            pl.BlockSpec((_D, _TBLK), lambda i: (0, i)),
            pl.BlockSpec(
                (_D, _TBLK),
                lambda i: (0, jnp.minimum(i + _TGRID, _N // _TBLK))),
        ],
        out_specs=pl.BlockSpec((_TBLK, 128), lambda i: (i, 0)),
        out_shape=jax.ShapeDtypeStruct((_PK, 128), jnp.float32),
    )(node_t, node_t)


def _sc_body(eg_hbm, eh_hbm, ug_hbm, uh_hbm, pi_hbm, node2, edge_t, out_hbm,
             egv, ehv, ugv, uhv, piv, edge_v,
             ebufa, ubufa, ebufb, ubufb,
             out_v, sema, semb):
    c = lax.axis_index("c")
    s = lax.axis_index("s")
    wid = s * 2 + c
    base = wid * _BPW

    pltpu.sync_copy(eg_hbm.at[wid], egv)
    pltpu.sync_copy(eh_hbm.at[wid], ehv)
    pltpu.sync_copy(ug_hbm.at[wid], ugv)
    pltpu.sync_copy(uh_hbm.at[wid], uhv)
    pltpu.sync_copy(pi_hbm.at[wid], piv)
    pltpu.sync_copy(edge_t, edge_v)

    bufs = [(ebufa, ubufa, sema), (ebufb, ubufb, semb)]

    def issue(j):
        eb_, ub_, sem = bufs[j % 2]
        return [
            pltpu.async_copy(node2.at[egv.at[j, 0]], eb_, sem),
            pltpu.async_copy(node2.at[ugv.at[j, 0]], ub_, sem),
        ]

    iota = lax.iota(jnp.int32, 16)
    jvecs = [jnp.full((16,), j, jnp.int32) for j in range(_NCH)]
    zero16 = jnp.zeros((16,), jnp.int32)

    def compute(j):
        eb_, ub_, _ = bufs[j % 2]
        jvec = jvecs[j]
        for g in range(_CH // 16):
            rows = g * 16 + iota
            he = plsc.load_gather(ehv, [jvec, zero16, rows])
            hu = plsc.load_gather(uhv, [jvec, zero16, rows])
            pp = plsc.load_gather(piv, [jvec, zero16, rows])
            z = jnp.zeros((16,), jnp.float32)

            def jj_body(jj, accs):
                a0, a1, a2, a3 = accs
                new = []
                for k in range(4):
                    f = jj * 4 + k
                    fv = jnp.full((16,), f, jnp.int32)
                    ev = plsc.load_gather(eb_, [rows, he + f])
                    uv = plsc.load_gather(ub_, [rows, hu + f])
                    pv = plsc.load_gather(edge_v, [fv, pp])
                    new.append(ev * pv * uv)
                return (a0 + new[0], a1 + new[1], a2 + new[2], a3 + new[3])

            a0, a1, a2, a3 = lax.fori_loop(0, _D // 4, jj_body, (z, z, z, z))
            out_v[pl.ds(j * _CH + g * 16, 16)] = (a0 + a1) + (a2 + a3)

    prev = issue(0)
    for j in range(1, _NCH):
        nxt = issue(j)
        for cp in prev:
            cp.wait()
        compute(j - 1)
        prev = nxt
    for cp in prev:
        cp.wait()
    compute(_NCH - 1)

    pltpu.sync_copy(out_v, out_hbm.at[pl.ds(base, _BPW)])


def _sc_call(eg, eh, ug, uh, pi, node2, edge_t):
    mesh = plsc.VectorSubcoreMesh(core_axis_name="c", subcore_axis_name="s")
    fn = functools.partial(
        pl.kernel,
        out_type=jax.ShapeDtypeStruct((_B,), jnp.float32),
        mesh=mesh,
        compiler_params=pltpu.CompilerParams(needs_layout_passes=False),
        scratch_types=[
            pltpu.VMEM((_NCH, 1, _CH), jnp.int32),
            pltpu.VMEM((_NCH, 1, _CH), jnp.int32),
            pltpu.VMEM((_NCH, 1, _CH), jnp.int32),
            pltpu.VMEM((_NCH, 1, _CH), jnp.int32),
            pltpu.VMEM((_NCH, 1, _CH), jnp.int32),
            pltpu.VMEM((_D, _R), jnp.float32),
            pltpu.VMEM((_CH, 128), jnp.float32),
            pltpu.VMEM((_CH, 128), jnp.float32),
            pltpu.VMEM((_CH, 128), jnp.float32),
            pltpu.VMEM((_CH, 128), jnp.float32),
            pltpu.VMEM((_BPW,), jnp.float32),
            pltpu.SemaphoreType.DMA,
            pltpu.SemaphoreType.DMA,
        ],
    )(_sc_body)
    return fn(eg, eh, ug, uh, pi, node2, edge_t)


def kernel(e_idc, p_idc, u_idc, node_embeddings, edge_embeddings):
    e = e_idc.astype(jnp.int32)
    p = p_idc.astype(jnp.int32)
    u = u_idc.astype(jnp.int32)
    shp = (_NW, _NCH, 1, _CH)
    eg = jnp.where(e < _PK, e, e - _PK).reshape(shp)
    eh = jnp.where(e < _PK, 0, _D).astype(jnp.int32).reshape(shp)
    ug = jnp.where(u < _PK, u, u - _PK).reshape(shp)
    uh = jnp.where(u < _PK, 0, _D).astype(jnp.int32).reshape(shp)
    pi = p.reshape(shp)
    node2 = _tpack(node_embeddings.T)
    return _sc_call(eg, eh, ug, uh, pi, node2, edge_embeddings.T)
